# baseline (speedup 1.0000x reference)
import jax
import jax.numpy as jnp
from jax import lax
from jax.experimental import pallas as pl
from jax.experimental.pallas import tpu as pltpu
from jax.experimental.pallas import tpu_sc as plsc

N_GENES = 50000
EMB = 64
N_EDGES = 800000
N_COND = 2048
BATCH = 4096
MAX_G = 5

NC, NS, L = 2, 16, 16
NW = NC * NS
ZSLOT = N_COND * MAX_G
ACC_ROWS = ZSLOT + L
ACC_STRIPE = ACC_ROWS // NS
MAPW = N_GENES // 2
CHUNK = 256
NBLK = CHUNK // L
EDGES_PAD = 819200
EPW = EDGES_PAD // NW
NCHUNK = EPW // CHUNK
CPT = N_COND // NS
PP = 16


def _decode(map16, genes):
    w = plsc.load_gather(map16, [lax.shift_right_logical(genes, 1)])
    half = jnp.where((genes & 1) == 1, lax.shift_right_logical(w, 16), w)
    return (half & 0xFFFF) - 1


def _sc_body(edata_hbm, mat_hbm, emb_hbm, zmap_hbm, z_hbm,
             out_hbm,
             map16, mat_v,
             ebuf_a, colsc_a, valsc_a, slotsc_a, rowbuf_a,
             ebuf_b, colsc_b, valsc_b, slotsc_b, rowbuf_b,
             slots3, gbuf, pooled_v, acc,
             esem, gsem_a, gsem_b, ssem_a, ssem_b):
    cid = lax.axis_index("c")
    sid = lax.axis_index("s")
    wid = cid * NS + sid
    iota = lax.iota(jnp.int32, L)

    pltpu.sync_copy(zmap_hbm, map16)
    pltpu.sync_copy(mat_hbm, mat_v)
    pltpu.sync_copy(z_hbm, acc.at[pl.ds(sid * ACC_STRIPE, ACC_STRIPE)])

    for par in (0, 1):
        @pl.loop(0, N_COND * MAX_G // L)
        def _(i):
            g = mat_v[pl.ds(i * L, L)]
            gs = jnp.maximum(g, 0)
            widx = lax.shift_right_logical(gs, 1)
            old = plsc.load_gather(map16, [widx])
            enc = iota + (i * L + 1)
            if par == 0:
                neww = (old & jnp.int32(-65536)) | enc
            else:
                neww = (old & jnp.int32(0xFFFF)) | lax.shift_left(enc, 16)
            plsc.store_scatter(map16, [widx], neww,
                               mask=(g >= 0) & ((g & 1) == par))

    plsc.subcore_barrier()

    cbase = wid * NCHUNK

    def fire_gathers(colsc_p, rowbuf_p, gsem_p, nblk):
        @pl.loop(0, NBLK)
        def _(b):
            @pl.when(b < nblk)
            def _():
                pltpu.async_copy(emb_hbm.at[colsc_p.at[pl.ds(b * L, L)]],
                                 rowbuf_p.at[pl.ds(b * L, L)], gsem_p)

    def s1(ch, ebuf_p, colsc_p, valsc_p, slotsc_p, rowbuf_p, gsem_p):
        pltpu.make_async_copy(edata_hbm.at[cbase + ch], ebuf_p, esem).wait()

        def compact(j, w):
            off = pl.ds(j * L, L)
            s16 = _decode(map16, ebuf_p[0, off])
            m = s16 >= 0
            mi = m.astype(jnp.int32)
            pos = w + jnp.cumsum(mi) - 1
            plsc.store_scatter(colsc_p, [pos], ebuf_p[1, off], mask=m)
            plsc.store_scatter(valsc_p, [pos],
                               plsc.bitcast(ebuf_p[2, off], jnp.float32),
                               mask=m)
            plsc.store_scatter(slotsc_p, [pos // L, pos % L], s16, mask=m)
            return w + jnp.sum(mi)

        n = lax.fori_loop(0, NBLK, compact, jnp.int32(0))
        padidx = n + iota
        plsc.store_scatter(colsc_p, [padidx], jnp.zeros((L,), jnp.int32))
        plsc.store_scatter(valsc_p, [padidx], jnp.zeros((L,), jnp.float32))
        plsc.store_scatter(slotsc_p, [padidx // L, padidx % L],
                           jnp.full((L,), ZSLOT, jnp.int32))
        nblk = (n + L - 1) // L
        fire_gathers(colsc_p, rowbuf_p, gsem_p, nblk)
        return nblk

    def s2(colsc_p, valsc_p, slotsc_p, rowbuf_p, gsem_p, ssem_p, nblk):
        @pl.loop(0, NBLK)
        def _(b):
            @pl.when(b < nblk)
            def _():
                pltpu.make_async_copy(
                    emb_hbm.at[colsc_p.at[pl.ds(b * L, L)]],
                    rowbuf_p.at[pl.ds(b * L, L)], gsem_p).wait()
                for j in range(L):
                    r = b * L + j
                    vv = plsc.load_gather(valsc_p,
                                          [jnp.full((L,), r, jnp.int32)])
                    for q in range(EMB // L):
                        sl = pl.ds(q * L, L)
                        rowbuf_p[r, sl] = rowbuf_p[r, sl] * vv
                pltpu.async_copy(rowbuf_p.at[pl.ds(b * L, L)],
                                 acc.at[slotsc_p.at[b]], ssem_p, add=True)

        @pl.loop(0, NBLK)
        def _(b):
            @pl.when(b < nblk)
            def _():
                pltpu.make_async_copy(rowbuf_p.at[pl.ds(b * L, L)],
                                      acc.at[slotsc_p.at[b]], ssem_p).wait()

    bufs_a = (colsc_a, valsc_a, slotsc_a, rowbuf_a, gsem_a)
    bufs_b = (colsc_b, valsc_b, slotsc_b, rowbuf_b, gsem_b)

    pltpu.async_copy(edata_hbm.at[cbase], ebuf_a, esem)

    def body2(k, nb_last):
        ch0 = 2 * k
        nb_a = s1(ch0, ebuf_a, *bufs_a)
        pltpu.async_copy(edata_hbm.at[cbase + ch0 + 1], ebuf_b, esem)

        @pl.when(k > 0)
        def _():
            s2(*bufs_b, ssem_b, nb_last)

        nb_b = s1(ch0 + 1, ebuf_b, *bufs_b)

        @pl.when(k + 1 < NCHUNK // 2)
        def _():
            pltpu.async_copy(edata_hbm.at[cbase + ch0 + 2], ebuf_a, esem)

        s2(*bufs_a, ssem_a, nb_a)
        return nb_b

    nb_fin = lax.fori_loop(0, NCHUNK // 2, body2, jnp.int32(0))
    s2(*bufs_b, ssem_b, nb_fin)

    plsc.subcore_barrier()

    @pl.loop(0, CPT // PP)
    def _(h):
        c0 = sid * CPT + h * PP
        sbase = c0 * MAX_G

        @pl.loop(0, PP * MAX_G // L)
        def _(i):
            g = mat_v[pl.ds(sbase + i * L, L)]
            s = _decode(map16, jnp.maximum(g, 0))
            slots3[pl.ds(i * L, L)] = jnp.where(g >= 0, s, ZSLOT)

        pltpu.sync_copy(acc.at[slots3], gbuf)

        @pl.loop(0, PP)
        def _(cc):
            b5 = cc * MAX_G
            for q in range(EMB // L):
                sl = pl.ds(q * L, L)
                ssum = gbuf[b5, sl]
                for j in range(1, MAX_G):
                    ssum = ssum + gbuf[b5 + j, sl]
                pooled_v[cc, sl] = ssum

        pltpu.sync_copy(pooled_v, out_hbm.at[cid, pl.ds(c0, PP)])


_sc_mesh = plsc.VectorSubcoreMesh(core_axis_name="c", subcore_axis_name="s")
_sc_params = pltpu.CompilerParams(needs_layout_passes=False,
                                  use_tc_tiling_on_sc=False)

_chunk_bufs = [
    pltpu.VMEM((3, CHUNK), jnp.int32),
    pltpu.VMEM((CHUNK + L,), jnp.int32),
    pltpu.VMEM((CHUNK + L,), jnp.float32),
    pltpu.VMEM((NBLK + 1, L), jnp.int32),
    pltpu.VMEM((CHUNK, EMB), jnp.float32),
]

_agg_pool = pl.kernel(
    _sc_body,
    out_type=jax.ShapeDtypeStruct((NC, N_COND, EMB), jnp.float32),
    mesh=_sc_mesh,
    compiler_params=_sc_params,
    scratch_types=[
        pltpu.VMEM((MAPW,), jnp.int32),
        pltpu.VMEM((N_COND * MAX_G,), jnp.int32),
        *_chunk_bufs,
        *_chunk_bufs,
        pltpu.VMEM((PP * MAX_G,), jnp.int32),
        pltpu.VMEM((PP * MAX_G, EMB), jnp.float32),
        pltpu.VMEM((PP, EMB), jnp.float32),
        pltpu.VMEM_SHARED((ACC_ROWS, EMB), jnp.float32),
        pltpu.SemaphoreType.DMA,
        pltpu.SemaphoreType.DMA,
        pltpu.SemaphoreType.DMA,
        pltpu.SemaphoreType.DMA,
        pltpu.SemaphoreType.DMA,
    ],
)


def _tc_body(pp_ref, gnn_ref, w1_ref, b1_ref, w2_ref, b2_ref, mask_ref,
             o_ref):
    p = pp_ref[0] + pp_ref[1]
    summed = jnp.dot(p, gnn_ref[...], preferred_element_type=jnp.float32)
    h = jnp.maximum(
        jnp.dot(summed, w1_ref[...], preferred_element_type=jnp.float32)
        + b1_ref[...], 0.0)
    h = jnp.maximum(
        jnp.dot(h, w2_ref[...], preferred_element_type=jnp.float32)
        + b2_ref[...], 0.0)
    ng = jnp.sum(mask_ref[...], axis=1, keepdims=True)
    o_ref[...] = jnp.where(ng == 0.0, 0.0, jnp.where(ng == 1.0, summed, h))


_mlp = pl.pallas_call(
    _tc_body,
    out_shape=jax.ShapeDtypeStruct((N_COND, EMB), jnp.float32),
)


def _gat_body(tab_hbm, idx_hbm, out_hbm, idx_v, row_v, sem):
    wid = lax.axis_index("c") * NS + lax.axis_index("s")
    base = wid * (BATCH // NW)
    pltpu.sync_copy(idx_hbm.at[pl.ds(base, BATCH // NW)], idx_v)
    pltpu.async_copy(tab_hbm.at[idx_v], row_v, sem).wait()
    pltpu.sync_copy(row_v, out_hbm.at[pl.ds(base, BATCH // NW)])


_expand = pl.kernel(
    _gat_body,
    out_type=jax.ShapeDtypeStruct((BATCH, EMB), jnp.float32),
    mesh=_sc_mesh,
    compiler_params=_sc_params,
    scratch_types=[
        pltpu.VMEM((BATCH // NW,), jnp.int32),
        pltpu.VMEM((BATCH // NW, EMB), jnp.float32),
        pltpu.SemaphoreType.DMA,
    ],
)


def kernel(inputs, pert_embedding, gnn_kernel, mlp_w1, mlp_b1, mlp_w2, mlp_b2,
           adj_rows, adj_cols, adj_vals, cond_gene_matrix, cond_gene_mask):
    pad = EDGES_PAD - N_EDGES
    rows_p = jnp.concatenate([adj_rows, jnp.zeros((pad,), jnp.int32)])
    cols_p = jnp.concatenate([adj_cols, jnp.zeros((pad,), jnp.int32)])
    vals_p = jnp.concatenate(
        [lax.bitcast_convert_type(adj_vals, jnp.int32),
         jnp.zeros((pad,), jnp.int32)])
    edata = jnp.stack([rows_p.reshape(-1, CHUNK), cols_p.reshape(-1, CHUNK),
                       vals_p.reshape(-1, CHUNK)], axis=1)
    mat_flat = cond_gene_matrix.reshape(-1)
    zmap = jnp.zeros((MAPW,), jnp.int32)
    zrows = jnp.zeros((ACC_STRIPE, EMB), jnp.float32)

    pooled_partial = _agg_pool(edata, mat_flat, pert_embedding, zmap, zrows)
    mask8 = jnp.pad(cond_gene_mask, ((0, 0), (0, 3)))
    out_cond = _mlp(pooled_partial, gnn_kernel, mlp_w1,
                    mlp_b1.reshape(1, EMB), mlp_w2, mlp_b2.reshape(1, EMB),
                    mask8)
    return _expand(out_cond, inputs.astype(jnp.int32))


# speedup vs baseline: 2.3722x; 2.3722x over previous
import jax
import jax.numpy as jnp
from jax import lax
from jax.experimental import pallas as pl
from jax.experimental.pallas import tpu as pltpu
from jax.experimental.pallas import tpu_sc as plsc

N_GENES = 50000
EMB = 64
N_EDGES = 800000
N_COND = 2048
BATCH = 4096
MAX_G = 5

NC, NS, L = 2, 16, 16
NW = NC * NS
ZSLOT = N_COND * MAX_G
ACC_ROWS = ZSLOT + L
ACC_STRIPE = ACC_ROWS // NS
MAPW = N_GENES // 2
CHUNK = 256
NBLK = CHUNK // L
EDGES_PAD = 819200
EPW = EDGES_PAD // NW
NCHUNK = EPW // CHUNK
RING = 1024
BATCH_R = 128
NB = RING // BATCH_R
CPT = N_COND // NS
PP = 16


def _decode(map16, genes):
    w = plsc.load_gather(map16, [lax.shift_right_logical(genes, 1)])
    half = jnp.where((genes & 1) == 1, lax.shift_right_logical(w, 16), w)
    return (half & 0xFFFF) - 1


def _sc_body(edata_hbm, mat_hbm, emb_hbm, zmap_hbm, z_hbm,
             out_hbm,
             map16, mat_v, ebuf, colsr, valsr, slotsr, rowbuf, bslots,
             slots3, gbuf, pooled_v, acc,
             esem, gsem_a, gsem_b, ssem_a, ssem_b):
    cid = lax.axis_index("c")
    sid = lax.axis_index("s")
    wid = cid * NS + sid
    iota = lax.iota(jnp.int32, L)

    pltpu.sync_copy(zmap_hbm, map16)
    pltpu.sync_copy(mat_hbm, mat_v)
    pltpu.sync_copy(z_hbm, acc.at[pl.ds(sid * ACC_STRIPE, ACC_STRIPE)])

    for par in (0, 1):
        @pl.loop(0, N_COND * MAX_G // L)
        def _(i):
            g = mat_v[pl.ds(i * L, L)]
            gs = jnp.maximum(g, 0)
            widx = lax.shift_right_logical(gs, 1)
            old = plsc.load_gather(map16, [widx])
            enc = iota + (i * L + 1)
            if par == 0:
                neww = (old & jnp.int32(-65536)) | enc
            else:
                neww = (old & jnp.int32(0xFFFF)) | lax.shift_left(enc, 16)
            plsc.store_scatter(map16, [widx], neww,
                               mask=(g >= 0) & ((g & 1) == par))

    plsc.subcore_barrier()

    cbase = wid * NCHUNK
    c15 = jnp.full((L,), L - 1, jnp.int32)

    def gat_desc(bat, par, sem):
        roff = (bat % NB) * BATCH_R
        return pltpu.make_async_copy(
            emb_hbm.at[colsr.at[pl.ds(roff, BATCH_R)]],
            rowbuf.at[pl.ds(par * BATCH_R, BATCH_R)],
            sem)

    def sca_desc(bat, par, sem):
        return pltpu.make_async_copy(
            rowbuf.at[pl.ds(par * BATCH_R, BATCH_R)],
            acc.at[bslots.at[par]],
            sem)

    def fire_body(F, par, sem_g, sem_s):
        @pl.when(F >= 2)
        def _():
            sca_desc(F - 2, par, sem_s).wait()
        roff = (F % NB) * BATCH_R

        @pl.loop(0, BATCH_R // L)
        def _(g):
            bslots[par, pl.ds(g * L, L)] = slotsr[pl.ds(roff + g * L, L)]

        gat_desc(F, par, sem_g).start()

    def fire_if(F, pred):
        @pl.when(pred & (F % 2 == 0))
        def _():
            fire_body(F, 0, gsem_a, ssem_a)

        @pl.when(pred & (F % 2 == 1))
        def _():
            fire_body(F, 1, gsem_b, ssem_b)

        return jnp.where(pred, F + 1, F)

    def process_body(C, par, sem_g, sem_s):
        gat_desc(C, par, sem_g).wait()
        roff = (C % NB) * BATCH_R
        rbase = par * BATCH_R

        @pl.loop(0, BATCH_R // L)
        def _(g):
            for j in range(L):
                vv = plsc.load_gather(
                    valsr, [jnp.full((L,), roff + g * L + j, jnp.int32)])
                r = rbase + g * L + j
                for q in range(EMB // L):
                    sl = pl.ds(q * L, L)
                    rowbuf[r, sl] = rowbuf[r, sl] * vv

        pltpu.async_copy(rowbuf.at[pl.ds(par * BATCH_R, BATCH_R)],
                         acc.at[bslots.at[par]], sem_s, add=True)

    def process_if(C, pred):
        @pl.when(pred & (C % 2 == 0))
        def _():
            process_body(C, 0, gsem_a, ssem_a)

        @pl.when(pred & (C % 2 == 1))
        def _():
            process_body(C, 1, gsem_b, ssem_b)

        return jnp.where(pred, C + 1, C)

    pltpu.async_copy(edata_hbm.at[cbase], ebuf.at[0], esem)

    def chunk_body(ch, carry):
        W, F, C = carry
        par = ch % 2
        pltpu.make_async_copy(edata_hbm.at[cbase + ch], ebuf.at[par],
                              esem).wait()

        @pl.when(ch + 1 < NCHUNK)
        def _():
            pltpu.async_copy(edata_hbm.at[cbase + ch + 1],
                             ebuf.at[(ch + 1) % 2], esem)

        def compact(j, w):
            off = pl.ds(j * L, L)
            s16 = _decode(map16, ebuf[par, 0, off])
            m = s16 >= 0
            mi = m.astype(jnp.int32)
            pos = w + jnp.cumsum(mi) - 1
            rpos = pos & (RING - 1)
            plsc.store_scatter(colsr, [rpos], ebuf[par, 1, off], mask=m)
            plsc.store_scatter(valsr, [rpos],
                               plsc.bitcast(ebuf[par, 2, off], jnp.float32),
                               mask=m)
            plsc.store_scatter(slotsr, [rpos], s16, mask=m)
            return w + jnp.sum(mi)

        W = lax.fori_loop(0, NBLK, compact, W)

        C = process_if(C, C < F)
        F = fire_if(F, (W - F * BATCH_R >= BATCH_R) & (F < C + 2))
        C = process_if(C, C < F - 1)
        F = fire_if(F, (W - F * BATCH_R >= BATCH_R) & (F < C + 2))
        return W, F, C

    W, F, C = lax.fori_loop(0, NCHUNK, chunk_body,
                            (jnp.int32(0), jnp.int32(0), jnp.int32(0)))

    Wp = (W + BATCH_R - 1) & ~(BATCH_R - 1)

    @pl.loop(0, BATCH_R // L)
    def _(g):
        pos = W + g * L + iota
        m = pos < Wp
        rpos = pos & (RING - 1)
        plsc.store_scatter(colsr, [rpos], jnp.zeros((L,), jnp.int32), mask=m)
        plsc.store_scatter(valsr, [rpos], jnp.zeros((L,), jnp.float32),
                           mask=m)
        plsc.store_scatter(slotsr, [rpos], jnp.full((L,), ZSLOT, jnp.int32),
                           mask=m)

    C = process_if(C, C < F)
    F = fire_if(F, (Wp - F * BATCH_R >= BATCH_R) & (F < C + 2))
    C = process_if(C, C < F)
    C = process_if(C, C < F)

    for back in (2, 1):
        @pl.when((C >= back) & ((C - back) % 2 == 0))
        def _():
            sca_desc(C - back, 0, ssem_a).wait()

        @pl.when((C >= back) & ((C - back) % 2 == 1))
        def _():
            sca_desc(C - back, 1, ssem_b).wait()

    plsc.subcore_barrier()

    @pl.loop(0, CPT // PP)
    def _(h):
        c0 = sid * CPT + h * PP
        sbase = c0 * MAX_G

        @pl.loop(0, PP * MAX_G // L)
        def _(i):
            g = mat_v[pl.ds(sbase + i * L, L)]
            s = _decode(map16, jnp.maximum(g, 0))
            slots3[pl.ds(i * L, L)] = jnp.where(g >= 0, s, ZSLOT)

        pltpu.sync_copy(acc.at[slots3], gbuf)

        @pl.loop(0, PP)
        def _(cc):
            b5 = cc * MAX_G
            for q in range(EMB // L):
                sl = pl.ds(q * L, L)
                ssum = gbuf[b5, sl]
                for j in range(1, MAX_G):
                    ssum = ssum + gbuf[b5 + j, sl]
                pooled_v[cc, sl] = ssum

        pltpu.sync_copy(pooled_v, out_hbm.at[cid, pl.ds(c0, PP)])


_sc_mesh = plsc.VectorSubcoreMesh(core_axis_name="c", subcore_axis_name="s")
_sc_params = pltpu.CompilerParams(needs_layout_passes=False,
                                  use_tc_tiling_on_sc=False)

_agg_pool = pl.kernel(
    _sc_body,
    out_type=jax.ShapeDtypeStruct((NC, N_COND, EMB), jnp.float32),
    mesh=_sc_mesh,
    compiler_params=_sc_params,
    scratch_types=[
        pltpu.VMEM((MAPW,), jnp.int32),
        pltpu.VMEM((N_COND * MAX_G,), jnp.int32),
        pltpu.VMEM((2, 3, CHUNK), jnp.int32),
        pltpu.VMEM((RING,), jnp.int32),
        pltpu.VMEM((RING,), jnp.float32),
        pltpu.VMEM((RING,), jnp.int32),
        pltpu.VMEM((2 * BATCH_R, EMB), jnp.float32),
        pltpu.VMEM((2, BATCH_R), jnp.int32),
        pltpu.VMEM((PP * MAX_G,), jnp.int32),
        pltpu.VMEM((PP * MAX_G, EMB), jnp.float32),
        pltpu.VMEM((PP, EMB), jnp.float32),
        pltpu.VMEM_SHARED((ACC_ROWS, EMB), jnp.float32),
        pltpu.SemaphoreType.DMA,
        pltpu.SemaphoreType.DMA,
        pltpu.SemaphoreType.DMA,
        pltpu.SemaphoreType.DMA,
        pltpu.SemaphoreType.DMA,
    ],
)


def _tc_body(pp_ref, gnn_ref, w1_ref, b1_ref, w2_ref, b2_ref, mask_ref,
             o_ref):
    p = pp_ref[0] + pp_ref[1]
    summed = jnp.dot(p, gnn_ref[...], preferred_element_type=jnp.float32)
    h = jnp.maximum(
        jnp.dot(summed, w1_ref[...], preferred_element_type=jnp.float32)
        + b1_ref[...], 0.0)
    h = jnp.maximum(
        jnp.dot(h, w2_ref[...], preferred_element_type=jnp.float32)
        + b2_ref[...], 0.0)
    ng = jnp.sum(mask_ref[...], axis=1, keepdims=True)
    o_ref[...] = jnp.where(ng == 0.0, 0.0, jnp.where(ng == 1.0, summed, h))


_mlp = pl.pallas_call(
    _tc_body,
    out_shape=jax.ShapeDtypeStruct((N_COND, EMB), jnp.float32),
)


def _gat_body(tab_hbm, idx_hbm, out_hbm, idx_v, row_v, sem):
    wid = lax.axis_index("c") * NS + lax.axis_index("s")
    base = wid * (BATCH // NW)
    pltpu.sync_copy(idx_hbm.at[pl.ds(base, BATCH // NW)], idx_v)
    pltpu.async_copy(tab_hbm.at[idx_v], row_v, sem).wait()
    pltpu.sync_copy(row_v, out_hbm.at[pl.ds(base, BATCH // NW)])


_expand = pl.kernel(
    _gat_body,
    out_type=jax.ShapeDtypeStruct((BATCH, EMB), jnp.float32),
    mesh=_sc_mesh,
    compiler_params=_sc_params,
    scratch_types=[
        pltpu.VMEM((BATCH // NW,), jnp.int32),
        pltpu.VMEM((BATCH // NW, EMB), jnp.float32),
        pltpu.SemaphoreType.DMA,
    ],
)


def kernel(inputs, pert_embedding, gnn_kernel, mlp_w1, mlp_b1, mlp_w2, mlp_b2,
           adj_rows, adj_cols, adj_vals, cond_gene_matrix, cond_gene_mask):
    pad = EDGES_PAD - N_EDGES
    rows_p = jnp.concatenate([adj_rows, jnp.zeros((pad,), jnp.int32)])
    cols_p = jnp.concatenate([adj_cols, jnp.zeros((pad,), jnp.int32)])
    vals_p = jnp.concatenate(
        [lax.bitcast_convert_type(adj_vals, jnp.int32),
         jnp.zeros((pad,), jnp.int32)])
    edata = jnp.stack([rows_p.reshape(-1, CHUNK), cols_p.reshape(-1, CHUNK),
                       vals_p.reshape(-1, CHUNK)], axis=1)
    mat_flat = cond_gene_matrix.reshape(-1)
    zmap = jnp.zeros((MAPW,), jnp.int32)
    zrows = jnp.zeros((ACC_STRIPE, EMB), jnp.float32)

    pooled_partial = _agg_pool(edata, mat_flat, pert_embedding, zmap, zrows)
    mask8 = jnp.pad(cond_gene_mask, ((0, 0), (0, 3)))
    out_cond = _mlp(pooled_partial, gnn_kernel, mlp_w1,
                    mlp_b1.reshape(1, EMB), mlp_w2, mlp_b2.reshape(1, EMB),
                    mask8)
    return _expand(out_cond, inputs.astype(jnp.int32))


# speedup vs baseline: 2.5250x; 1.0644x over previous
import jax
import jax.numpy as jnp
from jax import lax
from jax.experimental import pallas as pl
from jax.experimental.pallas import tpu as pltpu
from jax.experimental.pallas import tpu_sc as plsc

N_GENES = 50000
EMB = 64
N_EDGES = 800000
N_COND = 2048
BATCH = 4096
MAX_G = 5

NC, NS, L = 2, 16, 16
NW = NC * NS
ZSLOT = N_COND * MAX_G
ACC_ROWS = ZSLOT + L
ACC_STRIPE = ACC_ROWS // NS
MAPW = N_GENES // 2
CHUNK = 256
NBLK = CHUNK // L
NCHUNK = N_EDGES // CHUNK
BIGW = NCHUNK % NW
CPW = NCHUNK // NW
RING = 1024
BATCH_R = 128
NB = RING // BATCH_R
CPT = N_COND // NS
PP = 16


def _decode(map16, genes):
    w = plsc.load_gather(map16, [lax.shift_right_logical(genes, 1)])
    half = jnp.where((genes & 1) == 1, lax.shift_right_logical(w, 16), w)
    return (half & 0xFFFF) - 1


def _sc_body(rows_hbm, cols_hbm, vals_hbm, mat_hbm, emb_hbm, zmap_hbm, z_hbm,
             out_hbm,
             map16, mat_v, rowsb, colsb, valsb, colsr, valsr, slotsr,
             rowbuf, bslots, slots3, gbuf, pooled_v, acc,
             esem, gsem_a, gsem_b, ssem_a, ssem_b):
    cid = lax.axis_index("c")
    sid = lax.axis_index("s")
    wid = cid * NS + sid
    iota = lax.iota(jnp.int32, L)

    pltpu.sync_copy(zmap_hbm, map16)
    pltpu.sync_copy(mat_hbm, mat_v)
    pltpu.sync_copy(z_hbm, acc.at[pl.ds(sid * ACC_STRIPE, ACC_STRIPE)])

    for par in (0, 1):
        @pl.loop(0, N_COND * MAX_G // L)
        def _(i):
            g = mat_v[pl.ds(i * L, L)]
            gs = jnp.maximum(g, 0)
            widx = lax.shift_right_logical(gs, 1)
            old = plsc.load_gather(map16, [widx])
            enc = iota + (i * L + 1)
            if par == 0:
                neww = (old & jnp.int32(-65536)) | enc
            else:
                neww = (old & jnp.int32(0xFFFF)) | lax.shift_left(enc, 16)
            plsc.store_scatter(map16, [widx], neww,
                               mask=(g >= 0) & ((g & 1) == par))

    plsc.subcore_barrier()

    nch = jnp.where(wid < BIGW, CPW + 1, CPW)

    def fire_edata(k, par):
        e0 = (wid + NW * k) * CHUNK
        pltpu.async_copy(rows_hbm.at[pl.ds(e0, CHUNK)], rowsb.at[par], esem)
        pltpu.async_copy(cols_hbm.at[pl.ds(e0, CHUNK)], colsb.at[par], esem)
        pltpu.async_copy(vals_hbm.at[pl.ds(e0, CHUNK)], valsb.at[par], esem)

    def wait_edata(k, par):
        e0 = (wid + NW * k) * CHUNK
        pltpu.make_async_copy(rows_hbm.at[pl.ds(e0, CHUNK)], rowsb.at[par],
                              esem).wait()
        pltpu.make_async_copy(cols_hbm.at[pl.ds(e0, CHUNK)], colsb.at[par],
                              esem).wait()
        pltpu.make_async_copy(vals_hbm.at[pl.ds(e0, CHUNK)], valsb.at[par],
                              esem).wait()

    def gat_desc(bat, par, sem):
        roff = (bat % NB) * BATCH_R
        return pltpu.make_async_copy(
            emb_hbm.at[colsr.at[pl.ds(roff, BATCH_R)]],
            rowbuf.at[pl.ds(par * BATCH_R, BATCH_R)],
            sem)

    def sca_desc(bat, par, sem):
        return pltpu.make_async_copy(
            rowbuf.at[pl.ds(par * BATCH_R, BATCH_R)],
            acc.at[bslots.at[par]],
            sem)

    def fire_body(F, par, sem_g, sem_s):
        @pl.when(F >= 2)
        def _():
            sca_desc(F - 2, par, sem_s).wait()
        roff = (F % NB) * BATCH_R

        @pl.loop(0, BATCH_R // L)
        def _(g):
            bslots[par, pl.ds(g * L, L)] = slotsr[pl.ds(roff + g * L, L)]

        gat_desc(F, par, sem_g).start()

    def fire_if(F, pred):
        @pl.when(pred & (F % 2 == 0))
        def _():
            fire_body(F, 0, gsem_a, ssem_a)

        @pl.when(pred & (F % 2 == 1))
        def _():
            fire_body(F, 1, gsem_b, ssem_b)

        return jnp.where(pred, F + 1, F)

    def process_body(C, par, sem_g, sem_s):
        gat_desc(C, par, sem_g).wait()
        roff = (C % NB) * BATCH_R
        rbase = par * BATCH_R

        @pl.loop(0, BATCH_R // L)
        def _(g):
            for j in range(L):
                vv = plsc.load_gather(
                    valsr, [jnp.full((L,), roff + g * L + j, jnp.int32)])
                r = rbase + g * L + j
                for q in range(EMB // L):
                    sl = pl.ds(q * L, L)
                    rowbuf[r, sl] = rowbuf[r, sl] * vv

        pltpu.async_copy(rowbuf.at[pl.ds(par * BATCH_R, BATCH_R)],
                         acc.at[bslots.at[par]], sem_s, add=True)

    def process_if(C, pred):
        @pl.when(pred & (C % 2 == 0))
        def _():
            process_body(C, 0, gsem_a, ssem_a)

        @pl.when(pred & (C % 2 == 1))
        def _():
            process_body(C, 1, gsem_b, ssem_b)

        return jnp.where(pred, C + 1, C)

    fire_edata(0, 0)

    def chunk_body(ch, carry):
        W, F, C = carry
        par = ch % 2
        wait_edata(ch, par)

        @pl.when(ch + 1 < nch)
        def _():
            fire_edata(ch + 1, (ch + 1) % 2)

        def compact(j, w):
            off = pl.ds(j * L, L)
            s16 = _decode(map16, rowsb[par, off])
            m = s16 >= 0
            mi = m.astype(jnp.int32)
            pos = w + jnp.cumsum(mi) - 1
            rpos = pos & (RING - 1)
            plsc.store_scatter(colsr, [rpos], colsb[par, off], mask=m)
            plsc.store_scatter(valsr, [rpos], valsb[par, off], mask=m)
            plsc.store_scatter(slotsr, [rpos], s16, mask=m)
            return w + jnp.sum(mi)

        W = lax.fori_loop(0, NBLK, compact, W, unroll=2)

        C = process_if(C, C < F)
        F = fire_if(F, (W - F * BATCH_R >= BATCH_R) & (F < C + 2))
        C = process_if(C, C < F - 1)
        F = fire_if(F, (W - F * BATCH_R >= BATCH_R) & (F < C + 2))
        return W, F, C

    W, F, C = lax.fori_loop(0, nch, chunk_body,
                            (jnp.int32(0), jnp.int32(0), jnp.int32(0)))

    Wp = (W + BATCH_R - 1) & ~(BATCH_R - 1)

    @pl.loop(0, BATCH_R // L)
    def _(g):
        pos = W + g * L + iota
        m = pos < Wp
        rpos = pos & (RING - 1)
        plsc.store_scatter(colsr, [rpos], jnp.zeros((L,), jnp.int32), mask=m)
        plsc.store_scatter(valsr, [rpos], jnp.zeros((L,), jnp.float32),
                           mask=m)
        plsc.store_scatter(slotsr, [rpos], jnp.full((L,), ZSLOT, jnp.int32),
                           mask=m)

    C = process_if(C, C < F)
    F = fire_if(F, (Wp - F * BATCH_R >= BATCH_R) & (F < C + 2))
    C = process_if(C, C < F)
    C = process_if(C, C < F)

    for back in (2, 1):
        @pl.when((C >= back) & ((C - back) % 2 == 0))
        def _():
            sca_desc(C - back, 0, ssem_a).wait()

        @pl.when((C >= back) & ((C - back) % 2 == 1))
        def _():
            sca_desc(C - back, 1, ssem_b).wait()

    plsc.subcore_barrier()

    @pl.loop(0, CPT // PP)
    def _(h):
        c0 = sid * CPT + h * PP
        sbase = c0 * MAX_G

        @pl.loop(0, PP * MAX_G // L)
        def _(i):
            g = mat_v[pl.ds(sbase + i * L, L)]
            s = _decode(map16, jnp.maximum(g, 0))
            slots3[pl.ds(i * L, L)] = jnp.where(g >= 0, s, ZSLOT)

        pltpu.sync_copy(acc.at[slots3], gbuf)

        @pl.loop(0, PP)
        def _(cc):
            b5 = cc * MAX_G
            for q in range(EMB // L):
                sl = pl.ds(q * L, L)
                ssum = gbuf[b5, sl]
                for j in range(1, MAX_G):
                    ssum = ssum + gbuf[b5 + j, sl]
                pooled_v[cc, sl] = ssum

        pltpu.sync_copy(pooled_v, out_hbm.at[cid, pl.ds(c0, PP)])


_sc_mesh = plsc.VectorSubcoreMesh(core_axis_name="c", subcore_axis_name="s")
_sc_params = pltpu.CompilerParams(needs_layout_passes=False,
                                  use_tc_tiling_on_sc=False)

_agg_pool = pl.kernel(
    _sc_body,
    out_type=jax.ShapeDtypeStruct((NC, N_COND, EMB), jnp.float32),
    mesh=_sc_mesh,
    compiler_params=_sc_params,
    scratch_types=[
        pltpu.VMEM((MAPW,), jnp.int32),
        pltpu.VMEM((N_COND * MAX_G,), jnp.int32),
        pltpu.VMEM((2, CHUNK), jnp.int32),
        pltpu.VMEM((2, CHUNK), jnp.int32),
        pltpu.VMEM((2, CHUNK), jnp.float32),
        pltpu.VMEM((RING,), jnp.int32),
        pltpu.VMEM((RING,), jnp.float32),
        pltpu.VMEM((RING,), jnp.int32),
        pltpu.VMEM((2 * BATCH_R, EMB), jnp.float32),
        pltpu.VMEM((2, BATCH_R), jnp.int32),
        pltpu.VMEM((PP * MAX_G,), jnp.int32),
        pltpu.VMEM((PP * MAX_G, EMB), jnp.float32),
        pltpu.VMEM((PP, EMB), jnp.float32),
        pltpu.VMEM_SHARED((ACC_ROWS, EMB), jnp.float32),
        pltpu.SemaphoreType.DMA,
        pltpu.SemaphoreType.DMA,
        pltpu.SemaphoreType.DMA,
        pltpu.SemaphoreType.DMA,
        pltpu.SemaphoreType.DMA,
    ],
)


def _tc_body(pp_ref, gnn_ref, w1_ref, b1_ref, w2_ref, b2_ref, mask_ref,
             o_ref):
    p = pp_ref[0] + pp_ref[1]
    summed = jnp.dot(p, gnn_ref[...], preferred_element_type=jnp.float32)
    h = jnp.maximum(
        jnp.dot(summed, w1_ref[...], preferred_element_type=jnp.float32)
        + b1_ref[...], 0.0)
    h = jnp.maximum(
        jnp.dot(h, w2_ref[...], preferred_element_type=jnp.float32)
        + b2_ref[...], 0.0)
    ng = jnp.sum(mask_ref[...], axis=1, keepdims=True)
    o_ref[...] = jnp.where(ng == 0.0, 0.0, jnp.where(ng == 1.0, summed, h))


_mlp = pl.pallas_call(
    _tc_body,
    out_shape=jax.ShapeDtypeStruct((N_COND, EMB), jnp.float32),
)


def _gat_body(tab_hbm, idx_hbm, out_hbm, idx_v, row_v, sem):
    wid = lax.axis_index("c") * NS + lax.axis_index("s")
    base = wid * (BATCH // NW)
    pltpu.sync_copy(idx_hbm.at[pl.ds(base, BATCH // NW)], idx_v)
    pltpu.async_copy(tab_hbm.at[idx_v], row_v, sem).wait()
    pltpu.sync_copy(row_v, out_hbm.at[pl.ds(base, BATCH // NW)])


_expand = pl.kernel(
    _gat_body,
    out_type=jax.ShapeDtypeStruct((BATCH, EMB), jnp.float32),
    mesh=_sc_mesh,
    compiler_params=_sc_params,
    scratch_types=[
        pltpu.VMEM((BATCH // NW,), jnp.int32),
        pltpu.VMEM((BATCH // NW, EMB), jnp.float32),
        pltpu.SemaphoreType.DMA,
    ],
)


def kernel(inputs, pert_embedding, gnn_kernel, mlp_w1, mlp_b1, mlp_w2, mlp_b2,
           adj_rows, adj_cols, adj_vals, cond_gene_matrix, cond_gene_mask):
    mat_flat = cond_gene_matrix.reshape(-1)
    zmap = jnp.zeros((MAPW,), jnp.int32)
    zrows = jnp.zeros((ACC_STRIPE, EMB), jnp.float32)

    pooled_partial = _agg_pool(adj_rows, adj_cols, adj_vals, mat_flat,
                               pert_embedding, zmap, zrows)
    mask8 = jnp.pad(cond_gene_mask, ((0, 0), (0, 3)))
    out_cond = _mlp(pooled_partial, gnn_kernel, mlp_w1,
                    mlp_b1.reshape(1, EMB), mlp_w2, mlp_b2.reshape(1, EMB),
                    mask8)
    return _expand(out_cond, inputs.astype(jnp.int32))


# speedup vs baseline: 2.5694x; 1.0176x over previous
import jax
import jax.numpy as jnp
from jax import lax
from jax.experimental import pallas as pl
from jax.experimental.pallas import tpu as pltpu
from jax.experimental.pallas import tpu_sc as plsc

N_GENES = 50000
EMB = 64
N_EDGES = 800000
N_COND = 2048
BATCH = 4096
MAX_G = 5

NC, NS, L = 2, 16, 16
NW = NC * NS
ZSLOT = N_COND * MAX_G
ACC_ROWS = ZSLOT + L
ACC_STRIPE = ACC_ROWS // NS
MAPW = N_GENES // 2
CHUNK = 256
NBLK = CHUNK // L
NCHUNK = N_EDGES // CHUNK
BIGW = NCHUNK % NW
CPW = NCHUNK // NW
RING = 1024
BATCH_R = 128
NB = RING // BATCH_R
CPT = N_COND // NS
PP = 16


def _decode(map16, genes):
    w = plsc.load_gather(map16, [lax.shift_right_logical(genes, 1)])
    half = jnp.where((genes & 1) == 1, lax.shift_right_logical(w, 16), w)
    return (half & 0xFFFF) - 1


def _sc_body(rows_hbm, cols_hbm, vals_hbm, mat_hbm, emb_hbm, zmap_hbm, z_hbm,
             out_hbm,
             map16, mat_v, rowsb, colsb, valsb, colsr, valsr, slotsr,
             rowbuf, bslots, slots3, gbuf, pooled_v, acc,
             esem, gsem_a, gsem_b, ssem_a, ssem_b):
    cid = lax.axis_index("c")
    sid = lax.axis_index("s")
    wid = cid * NS + sid
    iota = lax.iota(jnp.int32, L)

    pltpu.sync_copy(zmap_hbm, map16)
    pltpu.sync_copy(mat_hbm, mat_v)
    pltpu.sync_copy(z_hbm, acc.at[pl.ds(sid * ACC_STRIPE, ACC_STRIPE)])

    @pl.loop(0, N_COND * MAX_G // L)
    def _(i):
        g = mat_v[pl.ds(i * L, L)]
        widx = lax.shift_right_logical(jnp.maximum(g, 0), 1)
        plsc.store_scatter(map16, [widx], iota + (i * L + 1),
                           mask=(g >= 0) & ((g & 1) == 0))

    @pl.loop(0, N_COND * MAX_G // L)
    def _(i):
        g = mat_v[pl.ds(i * L, L)]
        widx = lax.shift_right_logical(jnp.maximum(g, 0), 1)
        old = plsc.load_gather(map16, [widx])
        enc = iota + (i * L + 1)
        neww = (old & jnp.int32(0xFFFF)) | lax.shift_left(enc, 16)
        plsc.store_scatter(map16, [widx], neww,
                           mask=(g >= 0) & ((g & 1) == 1))

    plsc.subcore_barrier()

    nch = jnp.where(wid < BIGW, CPW + 1, CPW)

    def fire_edata(k, par):
        e0 = (wid + NW * k) * CHUNK
        pltpu.async_copy(rows_hbm.at[pl.ds(e0, CHUNK)], rowsb.at[par], esem)
        pltpu.async_copy(cols_hbm.at[pl.ds(e0, CHUNK)], colsb.at[par], esem)
        pltpu.async_copy(vals_hbm.at[pl.ds(e0, CHUNK)], valsb.at[par], esem)

    def wait_edata(k, par):
        e0 = (wid + NW * k) * CHUNK
        pltpu.make_async_copy(rows_hbm.at[pl.ds(e0, CHUNK)], rowsb.at[par],
                              esem).wait()
        pltpu.make_async_copy(cols_hbm.at[pl.ds(e0, CHUNK)], colsb.at[par],
                              esem).wait()
        pltpu.make_async_copy(vals_hbm.at[pl.ds(e0, CHUNK)], valsb.at[par],
                              esem).wait()

    def gat_desc(bat, par, sem):
        roff = (bat % NB) * BATCH_R
        return pltpu.make_async_copy(
            emb_hbm.at[colsr.at[pl.ds(roff, BATCH_R)]],
            rowbuf.at[pl.ds(par * BATCH_R, BATCH_R)],
            sem)

    def sca_desc(bat, par, sem):
        return pltpu.make_async_copy(
            rowbuf.at[pl.ds(par * BATCH_R, BATCH_R)],
            acc.at[bslots.at[par]],
            sem)

    def fire_body(F, par, sem_g, sem_s):
        @pl.when(F >= 2)
        def _():
            sca_desc(F - 2, par, sem_s).wait()
        roff = (F % NB) * BATCH_R

        @pl.loop(0, BATCH_R // L)
        def _(g):
            bslots[par, pl.ds(g * L, L)] = slotsr[pl.ds(roff + g * L, L)]

        gat_desc(F, par, sem_g).start()

    def fire_if(F, pred):
        @pl.when(pred & (F % 2 == 0))
        def _():
            fire_body(F, 0, gsem_a, ssem_a)

        @pl.when(pred & (F % 2 == 1))
        def _():
            fire_body(F, 1, gsem_b, ssem_b)

        return jnp.where(pred, F + 1, F)

    def process_body(C, par, sem_g, sem_s):
        gat_desc(C, par, sem_g).wait()
        roff = (C % NB) * BATCH_R
        rbase = par * BATCH_R

        @pl.loop(0, BATCH_R // L)
        def _(g):
            for j in range(L):
                vv = plsc.load_gather(
                    valsr, [jnp.full((L,), roff + g * L + j, jnp.int32)])
                r = rbase + g * L + j
                for q in range(EMB // L):
                    sl = pl.ds(q * L, L)
                    rowbuf[r, sl] = rowbuf[r, sl] * vv

        pltpu.async_copy(rowbuf.at[pl.ds(par * BATCH_R, BATCH_R)],
                         acc.at[bslots.at[par]], sem_s, add=True)

    def process_if(C, pred):
        @pl.when(pred & (C % 2 == 0))
        def _():
            process_body(C, 0, gsem_a, ssem_a)

        @pl.when(pred & (C % 2 == 1))
        def _():
            process_body(C, 1, gsem_b, ssem_b)

        return jnp.where(pred, C + 1, C)

    fire_edata(0, 0)
    fire_edata(1, 1)

    def chunk_body(ch, carry):
        W, F, C = carry
        par = ch % 3
        wait_edata(ch, par)

        @pl.when(ch + 2 < nch)
        def _():
            fire_edata(ch + 2, (ch + 2) % 3)

        def compact(j, w):
            off = pl.ds(j * L, L)
            s16 = _decode(map16, rowsb[par, off])
            m = s16 >= 0
            mi = m.astype(jnp.int32)
            pos = w + jnp.cumsum(mi) - 1
            rpos = pos & (RING - 1)
            plsc.store_scatter(colsr, [rpos], colsb[par, off], mask=m)
            plsc.store_scatter(valsr, [rpos], valsb[par, off], mask=m)
            plsc.store_scatter(slotsr, [rpos], s16, mask=m)
            return w + jnp.sum(mi)

        W = lax.fori_loop(0, NBLK, compact, W, unroll=2)

        C = process_if(C, C < F)
        F = fire_if(F, (W - F * BATCH_R >= BATCH_R) & (F < C + 2))
        C = process_if(C, C < F - 1)
        F = fire_if(F, (W - F * BATCH_R >= BATCH_R) & (F < C + 2))
        return W, F, C

    W, F, C = lax.fori_loop(0, nch, chunk_body,
                            (jnp.int32(0), jnp.int32(0), jnp.int32(0)))

    Wp = (W + BATCH_R - 1) & ~(BATCH_R - 1)

    @pl.loop(0, BATCH_R // L)
    def _(g):
        pos = W + g * L + iota
        m = pos < Wp
        rpos = pos & (RING - 1)
        plsc.store_scatter(colsr, [rpos], jnp.zeros((L,), jnp.int32), mask=m)
        plsc.store_scatter(valsr, [rpos], jnp.zeros((L,), jnp.float32),
                           mask=m)
        plsc.store_scatter(slotsr, [rpos], jnp.full((L,), ZSLOT, jnp.int32),
                           mask=m)

    C = process_if(C, C < F)
    F = fire_if(F, (Wp - F * BATCH_R >= BATCH_R) & (F < C + 2))
    C = process_if(C, C < F)
    C = process_if(C, C < F)

    for back in (2, 1):
        @pl.when((C >= back) & ((C - back) % 2 == 0))
        def _():
            sca_desc(C - back, 0, ssem_a).wait()

        @pl.when((C >= back) & ((C - back) % 2 == 1))
        def _():
            sca_desc(C - back, 1, ssem_b).wait()

    plsc.subcore_barrier()

    @pl.loop(0, CPT // PP)
    def _(h):
        c0 = sid * CPT + h * PP
        sbase = c0 * MAX_G

        @pl.loop(0, PP * MAX_G // L)
        def _(i):
            g = mat_v[pl.ds(sbase + i * L, L)]
            s = _decode(map16, jnp.maximum(g, 0))
            slots3[pl.ds(i * L, L)] = jnp.where(g >= 0, s, ZSLOT)

        pltpu.sync_copy(acc.at[slots3], gbuf)

        @pl.loop(0, PP)
        def _(cc):
            b5 = cc * MAX_G
            for q in range(EMB // L):
                sl = pl.ds(q * L, L)
                ssum = gbuf[b5, sl]
                for j in range(1, MAX_G):
                    ssum = ssum + gbuf[b5 + j, sl]
                pooled_v[cc, sl] = ssum

        pltpu.sync_copy(pooled_v, out_hbm.at[cid, pl.ds(c0, PP)])


_sc_mesh = plsc.VectorSubcoreMesh(core_axis_name="c", subcore_axis_name="s")
_sc_params = pltpu.CompilerParams(needs_layout_passes=False,
                                  use_tc_tiling_on_sc=False)

_agg_pool = pl.kernel(
    _sc_body,
    out_type=jax.ShapeDtypeStruct((NC, N_COND, EMB), jnp.float32),
    mesh=_sc_mesh,
    compiler_params=_sc_params,
    scratch_types=[
        pltpu.VMEM((MAPW,), jnp.int32),
        pltpu.VMEM((N_COND * MAX_G,), jnp.int32),
        pltpu.VMEM((3, CHUNK), jnp.int32),
        pltpu.VMEM((3, CHUNK), jnp.int32),
        pltpu.VMEM((3, CHUNK), jnp.float32),
        pltpu.VMEM((RING,), jnp.int32),
        pltpu.VMEM((RING,), jnp.float32),
        pltpu.VMEM((RING,), jnp.int32),
        pltpu.VMEM((2 * BATCH_R, EMB), jnp.float32),
        pltpu.VMEM((2, BATCH_R), jnp.int32),
        pltpu.VMEM((PP * MAX_G,), jnp.int32),
        pltpu.VMEM((PP * MAX_G, EMB), jnp.float32),
        pltpu.VMEM((PP, EMB), jnp.float32),
        pltpu.VMEM_SHARED((ACC_ROWS, EMB), jnp.float32),
        pltpu.SemaphoreType.DMA,
        pltpu.SemaphoreType.DMA,
        pltpu.SemaphoreType.DMA,
        pltpu.SemaphoreType.DMA,
        pltpu.SemaphoreType.DMA,
    ],
)


def _tc_body(pp_ref, gnn_ref, w1_ref, b1_ref, w2_ref, b2_ref, mask_ref,
             o_ref):
    p = pp_ref[0] + pp_ref[1]
    summed = jnp.dot(p, gnn_ref[...], preferred_element_type=jnp.float32)
    h = jnp.maximum(
        jnp.dot(summed, w1_ref[...], preferred_element_type=jnp.float32)
        + b1_ref[...], 0.0)
    h = jnp.maximum(
        jnp.dot(h, w2_ref[...], preferred_element_type=jnp.float32)
        + b2_ref[...], 0.0)
    ng = jnp.sum(mask_ref[...], axis=1, keepdims=True)
    o_ref[...] = jnp.where(ng == 0.0, 0.0, jnp.where(ng == 1.0, summed, h))


_mlp = pl.pallas_call(
    _tc_body,
    out_shape=jax.ShapeDtypeStruct((N_COND, EMB), jnp.float32),
)


def _gat_body(tab_hbm, idx_hbm, out_hbm, idx_v, row_v, sem):
    wid = lax.axis_index("c") * NS + lax.axis_index("s")
    base = wid * (BATCH // NW)
    pltpu.sync_copy(idx_hbm.at[pl.ds(base, BATCH // NW)], idx_v)
    pltpu.async_copy(tab_hbm.at[idx_v], row_v, sem).wait()
    pltpu.sync_copy(row_v, out_hbm.at[pl.ds(base, BATCH // NW)])


_expand = pl.kernel(
    _gat_body,
    out_type=jax.ShapeDtypeStruct((BATCH, EMB), jnp.float32),
    mesh=_sc_mesh,
    compiler_params=_sc_params,
    scratch_types=[
        pltpu.VMEM((BATCH // NW,), jnp.int32),
        pltpu.VMEM((BATCH // NW, EMB), jnp.float32),
        pltpu.SemaphoreType.DMA,
    ],
)


def kernel(inputs, pert_embedding, gnn_kernel, mlp_w1, mlp_b1, mlp_w2, mlp_b2,
           adj_rows, adj_cols, adj_vals, cond_gene_matrix, cond_gene_mask):
    mat_flat = cond_gene_matrix.reshape(-1)
    zmap = jnp.zeros((MAPW,), jnp.int32)
    zrows = jnp.zeros((ACC_STRIPE, EMB), jnp.float32)

    pooled_partial = _agg_pool(adj_rows, adj_cols, adj_vals, mat_flat,
                               pert_embedding, zmap, zrows)
    mask8 = jnp.pad(cond_gene_mask, ((0, 0), (0, 3)))
    out_cond = _mlp(pooled_partial, gnn_kernel, mlp_w1,
                    mlp_b1.reshape(1, EMB), mlp_w2, mlp_b2.reshape(1, EMB),
                    mask8)
    return _expand(out_cond, inputs.astype(jnp.int32))


# speedup vs baseline: 2.6026x; 1.0129x over previous
import jax
import jax.numpy as jnp
from jax import lax
from jax.experimental import pallas as pl
from jax.experimental.pallas import tpu as pltpu
from jax.experimental.pallas import tpu_sc as plsc

N_GENES = 50000
EMB = 64
N_EDGES = 800000
N_COND = 2048
BATCH = 4096
MAX_G = 5

NC, NS, L = 2, 16, 16
NW = NC * NS
ZSLOT = N_COND * MAX_G
ACC_ROWS = ZSLOT + L
ACC_STRIPE = ACC_ROWS // NS
MAPW = N_GENES // 2
CHUNK = 256
NBLK = CHUNK // L
NCHUNK = N_EDGES // CHUNK
BIGW = NCHUNK % NW
CPW = NCHUNK // NW
RING = 1024
BATCH_R = 128
NB = RING // BATCH_R
CPT = N_COND // NS
PP = 16


def _decode(map16, genes):
    w = plsc.load_gather(map16, [lax.shift_right_logical(genes, 1)])
    half = jnp.where((genes & 1) == 1, lax.shift_right_logical(w, 16), w)
    return (half & 0xFFFF) - 1


def _sc_body(rows_hbm, cols_hbm, vals_hbm, mat_hbm, emb_hbm, zmap_hbm, z_hbm,
             out_hbm,
             map16, mat_v, rowsb, colsb, valsb, colsr, valsr, slotsr,
             rowbuf, bslots, slots3, gbuf, pooled_v, acc,
             esem, gsem_a, gsem_b, ssem_a, ssem_b, g3sem, psem):
    cid = lax.axis_index("c")
    sid = lax.axis_index("s")
    wid = cid * NS + sid
    iota = lax.iota(jnp.int32, L)

    pltpu.sync_copy(zmap_hbm, map16)
    pltpu.sync_copy(mat_hbm, mat_v)
    pltpu.sync_copy(z_hbm, acc.at[pl.ds(sid * ACC_STRIPE, ACC_STRIPE)])

    @pl.loop(0, N_COND * MAX_G // L)
    def _(i):
        g = mat_v[pl.ds(i * L, L)]
        widx = lax.shift_right_logical(jnp.maximum(g, 0), 1)
        plsc.store_scatter(map16, [widx], iota + (i * L + 1),
                           mask=(g >= 0) & ((g & 1) == 0))

    @pl.loop(0, N_COND * MAX_G // L)
    def _(i):
        g = mat_v[pl.ds(i * L, L)]
        widx = lax.shift_right_logical(jnp.maximum(g, 0), 1)
        old = plsc.load_gather(map16, [widx])
        enc = iota + (i * L + 1)
        neww = (old & jnp.int32(0xFFFF)) | lax.shift_left(enc, 16)
        plsc.store_scatter(map16, [widx], neww,
                           mask=(g >= 0) & ((g & 1) == 1))

    plsc.subcore_barrier()

    nch = jnp.where(wid < BIGW, CPW + 1, CPW)

    def fire_edata(k, par):
        e0 = (wid + NW * k) * CHUNK
        pltpu.async_copy(rows_hbm.at[pl.ds(e0, CHUNK)], rowsb.at[par], esem)
        pltpu.async_copy(cols_hbm.at[pl.ds(e0, CHUNK)], colsb.at[par], esem)
        pltpu.async_copy(vals_hbm.at[pl.ds(e0, CHUNK)], valsb.at[par], esem)

    def wait_edata(k, par):
        e0 = (wid + NW * k) * CHUNK
        pltpu.make_async_copy(rows_hbm.at[pl.ds(e0, CHUNK)], rowsb.at[par],
                              esem).wait()
        pltpu.make_async_copy(cols_hbm.at[pl.ds(e0, CHUNK)], colsb.at[par],
                              esem).wait()
        pltpu.make_async_copy(vals_hbm.at[pl.ds(e0, CHUNK)], valsb.at[par],
                              esem).wait()

    def gat_desc(bat, par, sem):
        roff = (bat % NB) * BATCH_R
        return pltpu.make_async_copy(
            emb_hbm.at[colsr.at[pl.ds(roff, BATCH_R)]],
            rowbuf.at[pl.ds(par * BATCH_R, BATCH_R)],
            sem)

    def sca_desc(bat, par, sem):
        return pltpu.make_async_copy(
            rowbuf.at[pl.ds(par * BATCH_R, BATCH_R)],
            acc.at[bslots.at[par]],
            sem)

    def fire_body(F, par, sem_g, sem_s):
        @pl.when(F >= 2)
        def _():
            sca_desc(F - 2, par, sem_s).wait()
        roff = (F % NB) * BATCH_R

        @pl.loop(0, BATCH_R // L)
        def _(g):
            bslots[par, pl.ds(g * L, L)] = slotsr[pl.ds(roff + g * L, L)]

        gat_desc(F, par, sem_g).start()

    def fire_if(F, pred):
        @pl.when(pred & (F % 2 == 0))
        def _():
            fire_body(F, 0, gsem_a, ssem_a)

        @pl.when(pred & (F % 2 == 1))
        def _():
            fire_body(F, 1, gsem_b, ssem_b)

        return jnp.where(pred, F + 1, F)

    def process_body(C, par, sem_g, sem_s):
        gat_desc(C, par, sem_g).wait()
        roff = (C % NB) * BATCH_R
        rbase = par * BATCH_R

        @pl.loop(0, BATCH_R // L)
        def _(g):
            for j in range(L):
                vv = plsc.load_gather(
                    valsr, [jnp.full((L,), roff + g * L + j, jnp.int32)])
                r = rbase + g * L + j
                for q in range(EMB // L):
                    sl = pl.ds(q * L, L)
                    rowbuf[r, sl] = rowbuf[r, sl] * vv

        pltpu.async_copy(rowbuf.at[pl.ds(par * BATCH_R, BATCH_R)],
                         acc.at[bslots.at[par]], sem_s, add=True)

    def process_if(C, pred):
        @pl.when(pred & (C % 2 == 0))
        def _():
            process_body(C, 0, gsem_a, ssem_a)

        @pl.when(pred & (C % 2 == 1))
        def _():
            process_body(C, 1, gsem_b, ssem_b)

        return jnp.where(pred, C + 1, C)

    fire_edata(0, 0)
    fire_edata(1, 1)

    def chunk_body(ch, carry):
        W, F, C = carry
        par = ch % 3
        wait_edata(ch, par)

        @pl.when(ch + 2 < nch)
        def _():
            fire_edata(ch + 2, (ch + 2) % 3)

        def compact(j, w):
            off = pl.ds(j * L, L)
            s16 = _decode(map16, rowsb[par, off])
            m = s16 >= 0
            mi = m.astype(jnp.int32)
            pos = w + jnp.cumsum(mi) - 1
            rpos = pos & (RING - 1)
            plsc.store_scatter(colsr, [rpos], colsb[par, off], mask=m)
            plsc.store_scatter(valsr, [rpos], valsb[par, off], mask=m)
            plsc.store_scatter(slotsr, [rpos], s16, mask=m)
            return w + jnp.sum(mi)

        W = lax.fori_loop(0, NBLK, compact, W, unroll=2)

        C = process_if(C, C < F)
        F = fire_if(F, (W - F * BATCH_R >= BATCH_R) & (F < C + 2))
        C = process_if(C, C < F - 1)
        F = fire_if(F, (W - F * BATCH_R >= BATCH_R) & (F < C + 2))
        return W, F, C

    W, F, C = lax.fori_loop(0, nch, chunk_body,
                            (jnp.int32(0), jnp.int32(0), jnp.int32(0)))

    Wp = (W + BATCH_R - 1) & ~(BATCH_R - 1)

    @pl.loop(0, BATCH_R // L)
    def _(g):
        pos = W + g * L + iota
        m = pos < Wp
        rpos = pos & (RING - 1)
        plsc.store_scatter(colsr, [rpos], jnp.zeros((L,), jnp.int32), mask=m)
        plsc.store_scatter(valsr, [rpos], jnp.zeros((L,), jnp.float32),
                           mask=m)
        plsc.store_scatter(slotsr, [rpos], jnp.full((L,), ZSLOT, jnp.int32),
                           mask=m)

    C = process_if(C, C < F)
    F = fire_if(F, (Wp - F * BATCH_R >= BATCH_R) & (F < C + 2))
    C = process_if(C, C < F)
    C = process_if(C, C < F)

    for back in (2, 1):
        @pl.when((C >= back) & ((C - back) % 2 == 0))
        def _():
            sca_desc(C - back, 0, ssem_a).wait()

        @pl.when((C >= back) & ((C - back) % 2 == 1))
        def _():
            sca_desc(C - back, 1, ssem_b).wait()

    plsc.subcore_barrier()

    NP3 = CPT // PP
    PW = PP * MAX_G

    @pl.loop(0, CPT * MAX_G // L)
    def _(i):
        g = mat_v[pl.ds(sid * CPT * MAX_G + i * L, L)]
        s = _decode(map16, jnp.maximum(g, 0))
        slots3[pl.ds(i * L, L)] = jnp.where(g >= 0, s, ZSLOT)

    def p3_gat(h, par):
        return pltpu.make_async_copy(
            acc.at[slots3.at[pl.ds(h * PW, PW)]],
            gbuf.at[pl.ds(par * PW, PW)], g3sem)

    def p3_out(h, par):
        return pltpu.make_async_copy(
            pooled_v.at[pl.ds(par * PP, PP)],
            out_hbm.at[cid, pl.ds(sid * CPT + h * PP, PP)], psem)

    p3_gat(0, 0).start()
    p3_gat(1, 1).start()

    @pl.loop(0, NP3)
    def _(h):
        par = h % 2
        p3_gat(h, par).wait()

        @pl.when(h >= 2)
        def _():
            p3_out(h - 2, par).wait()

        @pl.loop(0, PP)
        def _(cc):
            b5 = par * PW + cc * MAX_G
            for q in range(EMB // L):
                sl = pl.ds(q * L, L)
                ssum = gbuf[b5, sl]
                for j in range(1, MAX_G):
                    ssum = ssum + gbuf[b5 + j, sl]
                pooled_v[par * PP + cc, sl] = ssum

        p3_out(h, par).start()

        @pl.when(h + 2 < NP3)
        def _():
            p3_gat(h + 2, par).start()

    p3_out(NP3 - 2, 0).wait()
    p3_out(NP3 - 1, 1).wait()


_sc_mesh = plsc.VectorSubcoreMesh(core_axis_name="c", subcore_axis_name="s")
_sc_params = pltpu.CompilerParams(needs_layout_passes=False,
                                  use_tc_tiling_on_sc=False)

_agg_pool = pl.kernel(
    _sc_body,
    out_type=jax.ShapeDtypeStruct((NC, N_COND, EMB), jnp.float32),
    mesh=_sc_mesh,
    compiler_params=_sc_params,
    scratch_types=[
        pltpu.VMEM((MAPW,), jnp.int32),
        pltpu.VMEM((N_COND * MAX_G,), jnp.int32),
        pltpu.VMEM((3, CHUNK), jnp.int32),
        pltpu.VMEM((3, CHUNK), jnp.int32),
        pltpu.VMEM((3, CHUNK), jnp.float32),
        pltpu.VMEM((RING,), jnp.int32),
        pltpu.VMEM((RING,), jnp.float32),
        pltpu.VMEM((RING,), jnp.int32),
        pltpu.VMEM((2 * BATCH_R, EMB), jnp.float32),
        pltpu.VMEM((2, BATCH_R), jnp.int32),
        pltpu.VMEM((CPT * MAX_G,), jnp.int32),
        pltpu.VMEM((2 * PP * MAX_G, EMB), jnp.float32),
        pltpu.VMEM((2 * PP, EMB), jnp.float32),
        pltpu.VMEM_SHARED((ACC_ROWS, EMB), jnp.float32),
        pltpu.SemaphoreType.DMA,
        pltpu.SemaphoreType.DMA,
        pltpu.SemaphoreType.DMA,
        pltpu.SemaphoreType.DMA,
        pltpu.SemaphoreType.DMA,
        pltpu.SemaphoreType.DMA,
        pltpu.SemaphoreType.DMA,
    ],
)


def _tc_body(pp_ref, gnn_ref, w1_ref, b1_ref, w2_ref, b2_ref, mask_ref,
             o_ref):
    p = pp_ref[0] + pp_ref[1]
    summed = jnp.dot(p, gnn_ref[...], preferred_element_type=jnp.float32)
    h = jnp.maximum(
        jnp.dot(summed, w1_ref[...], preferred_element_type=jnp.float32)
        + b1_ref[...], 0.0)
    h = jnp.maximum(
        jnp.dot(h, w2_ref[...], preferred_element_type=jnp.float32)
        + b2_ref[...], 0.0)
    ng = jnp.sum(mask_ref[...], axis=1, keepdims=True)
    o_ref[...] = jnp.where(ng == 0.0, 0.0, jnp.where(ng == 1.0, summed, h))


_mlp = pl.pallas_call(
    _tc_body,
    out_shape=jax.ShapeDtypeStruct((N_COND, EMB), jnp.float32),
)


def _gat_body(tab_hbm, idx_hbm, out_hbm, idx_v, row_v, sem):
    wid = lax.axis_index("c") * NS + lax.axis_index("s")
    base = wid * (BATCH // NW)
    pltpu.sync_copy(idx_hbm.at[pl.ds(base, BATCH // NW)], idx_v)
    pltpu.async_copy(tab_hbm.at[idx_v], row_v, sem).wait()
    pltpu.sync_copy(row_v, out_hbm.at[pl.ds(base, BATCH // NW)])


_expand = pl.kernel(
    _gat_body,
    out_type=jax.ShapeDtypeStruct((BATCH, EMB), jnp.float32),
    mesh=_sc_mesh,
    compiler_params=_sc_params,
    scratch_types=[
        pltpu.VMEM((BATCH // NW,), jnp.int32),
        pltpu.VMEM((BATCH // NW, EMB), jnp.float32),
        pltpu.SemaphoreType.DMA,
    ],
)


def kernel(inputs, pert_embedding, gnn_kernel, mlp_w1, mlp_b1, mlp_w2, mlp_b2,
           adj_rows, adj_cols, adj_vals, cond_gene_matrix, cond_gene_mask):
    mat_flat = cond_gene_matrix.reshape(-1)
    zmap = jnp.zeros((MAPW,), jnp.int32)
    zrows = jnp.zeros((ACC_STRIPE, EMB), jnp.float32)

    pooled_partial = _agg_pool(adj_rows, adj_cols, adj_vals, mat_flat,
                               pert_embedding, zmap, zrows)
    mask8 = jnp.pad(cond_gene_mask, ((0, 0), (0, 3)))
    out_cond = _mlp(pooled_partial, gnn_kernel, mlp_w1,
                    mlp_b1.reshape(1, EMB), mlp_w2, mlp_b2.reshape(1, EMB),
                    mask8)
    return _expand(out_cond, inputs.astype(jnp.int32))


# speedup vs baseline: 2.6139x; 1.0044x over previous
import jax
import jax.numpy as jnp
from jax import lax
from jax.experimental import pallas as pl
from jax.experimental.pallas import tpu as pltpu
from jax.experimental.pallas import tpu_sc as plsc

N_GENES = 50000
EMB = 64
N_EDGES = 800000
N_COND = 2048
BATCH = 4096
MAX_G = 5

NC, NS, L = 2, 16, 16
NW = NC * NS
ZSLOT = N_COND * MAX_G
ACC_ROWS = ZSLOT + L
ACC_STRIPE = ACC_ROWS // NS
MAPW = N_GENES // 2
CHUNK = 256
NBLK = CHUNK // L
NCHUNK = N_EDGES // CHUNK
BIGW = NCHUNK % NW
CPW = NCHUNK // NW
RING = 1024
BATCH_R = 128
NB = RING // BATCH_R
CPT = N_COND // NS
PP = 16


def _decode(map16, genes):
    w = plsc.load_gather(map16, [lax.shift_right_logical(genes, 1)])
    half = jnp.where((genes & 1) == 1, lax.shift_right_logical(w, 16), w)
    return (half & 0xFFFF) - 1


def _sc_body(rows_hbm, cols_hbm, vals_hbm, mat_hbm, emb_hbm, zmap_hbm, z_hbm,
             out_hbm,
             map16, mat_v, rowsb, colsb, valsb, colsr, valsr, slotsr,
             rowbuf, bslots, slots3, gbuf, pooled_v, acc,
             esem, gsem_a, gsem_b, ssem_a, ssem_b, g3sem, psem):
    cid = lax.axis_index("c")
    sid = lax.axis_index("s")
    wid = cid * NS + sid
    iota = lax.iota(jnp.int32, L)

    pltpu.sync_copy(zmap_hbm, map16)
    pltpu.sync_copy(mat_hbm, mat_v)
    pltpu.sync_copy(z_hbm, acc.at[pl.ds(sid * ACC_STRIPE, ACC_STRIPE)])

    @pl.loop(0, N_COND * MAX_G // L)
    def _(i):
        g = mat_v[pl.ds(i * L, L)]
        widx = lax.shift_right_logical(jnp.maximum(g, 0), 1)
        plsc.store_scatter(map16, [widx], iota + (i * L + 1),
                           mask=(g >= 0) & ((g & 1) == 0))

    @pl.loop(0, N_COND * MAX_G // L)
    def _(i):
        g = mat_v[pl.ds(i * L, L)]
        widx = lax.shift_right_logical(jnp.maximum(g, 0), 1)
        old = plsc.load_gather(map16, [widx])
        enc = iota + (i * L + 1)
        neww = (old & jnp.int32(0xFFFF)) | lax.shift_left(enc, 16)
        plsc.store_scatter(map16, [widx], neww,
                           mask=(g >= 0) & ((g & 1) == 1))

    plsc.subcore_barrier()

    nch = jnp.where(wid < BIGW, CPW + 1, CPW)

    def fire_edata(k, par):
        e0 = (wid + NW * k) * CHUNK
        pltpu.async_copy(rows_hbm.at[pl.ds(e0, CHUNK)], rowsb.at[par], esem)
        pltpu.async_copy(cols_hbm.at[pl.ds(e0, CHUNK)], colsb.at[par], esem)
        pltpu.async_copy(vals_hbm.at[pl.ds(e0, CHUNK)], valsb.at[par], esem)

    def wait_edata(k, par):
        e0 = (wid + NW * k) * CHUNK
        pltpu.make_async_copy(rows_hbm.at[pl.ds(e0, CHUNK)], rowsb.at[par],
                              esem).wait()
        pltpu.make_async_copy(cols_hbm.at[pl.ds(e0, CHUNK)], colsb.at[par],
                              esem).wait()
        pltpu.make_async_copy(vals_hbm.at[pl.ds(e0, CHUNK)], valsb.at[par],
                              esem).wait()

    def gat_desc(bat, par, sem):
        roff = (bat % NB) * BATCH_R
        return pltpu.make_async_copy(
            emb_hbm.at[colsr.at[pl.ds(roff, BATCH_R)]],
            rowbuf.at[pl.ds(par * BATCH_R, BATCH_R)],
            sem)

    def sca_desc(bat, par, sem):
        return pltpu.make_async_copy(
            rowbuf.at[pl.ds(par * BATCH_R, BATCH_R)],
            acc.at[bslots.at[par]],
            sem)

    def fire_body(F, par, sem_g, sem_s):
        @pl.when(F >= 2)
        def _():
            sca_desc(F - 2, par, sem_s).wait()
        roff = (F % NB) * BATCH_R

        @pl.loop(0, BATCH_R // L)
        def _(g):
            bslots[par, pl.ds(g * L, L)] = slotsr[pl.ds(roff + g * L, L)]

        gat_desc(F, par, sem_g).start()

    def fire_if(F, pred):
        @pl.when(pred & (F % 2 == 0))
        def _():
            fire_body(F, 0, gsem_a, ssem_a)

        @pl.when(pred & (F % 2 == 1))
        def _():
            fire_body(F, 1, gsem_b, ssem_b)

        return jnp.where(pred, F + 1, F)

    def process_body(C, par, sem_g, sem_s):
        gat_desc(C, par, sem_g).wait()
        roff = (C % NB) * BATCH_R
        rbase = par * BATCH_R

        @pl.loop(0, BATCH_R // L)
        def _(g):
            for j in range(L):
                vv = plsc.load_gather(
                    valsr, [jnp.full((L,), roff + g * L + j, jnp.int32)])
                r = rbase + g * L + j
                for q in range(EMB // L):
                    sl = pl.ds(q * L, L)
                    rowbuf[r, sl] = rowbuf[r, sl] * vv

        pltpu.async_copy(rowbuf.at[pl.ds(par * BATCH_R, BATCH_R)],
                         acc.at[bslots.at[par]], sem_s, add=True)

    def process_if(C, pred):
        @pl.when(pred & (C % 2 == 0))
        def _():
            process_body(C, 0, gsem_a, ssem_a)

        @pl.when(pred & (C % 2 == 1))
        def _():
            process_body(C, 1, gsem_b, ssem_b)

        return jnp.where(pred, C + 1, C)

    fire_edata(0, 0)
    fire_edata(1, 1)

    def chunk_body(ch, carry):
        W, F, C = carry
        par = ch % 3
        wait_edata(ch, par)

        @pl.when(ch + 2 < nch)
        def _():
            fire_edata(ch + 2, (ch + 2) % 3)

        def compact(j, w):
            off = pl.ds(j * L, L)
            s16 = _decode(map16, rowsb[par, off])
            m = s16 >= 0
            mi = m.astype(jnp.int32)
            pos = w + jnp.cumsum(mi) - 1
            rpos = pos & (RING - 1)
            plsc.store_scatter(colsr, [rpos], colsb[par, off], mask=m)
            plsc.store_scatter(valsr, [rpos], valsb[par, off], mask=m)
            plsc.store_scatter(slotsr, [rpos], s16, mask=m)
            return w + jnp.sum(mi)

        W = lax.fori_loop(0, NBLK, compact, W, unroll=4)

        C = process_if(C, C < F)
        F = fire_if(F, (W - F * BATCH_R >= BATCH_R) & (F < C + 2))
        C = process_if(C, C < F - 1)
        F = fire_if(F, (W - F * BATCH_R >= BATCH_R) & (F < C + 2))
        return W, F, C

    W, F, C = lax.fori_loop(0, nch, chunk_body,
                            (jnp.int32(0), jnp.int32(0), jnp.int32(0)))

    Wp = (W + BATCH_R - 1) & ~(BATCH_R - 1)

    @pl.loop(0, BATCH_R // L)
    def _(g):
        pos = W + g * L + iota
        m = pos < Wp
        rpos = pos & (RING - 1)
        plsc.store_scatter(colsr, [rpos], jnp.zeros((L,), jnp.int32), mask=m)
        plsc.store_scatter(valsr, [rpos], jnp.zeros((L,), jnp.float32),
                           mask=m)
        plsc.store_scatter(slotsr, [rpos], jnp.full((L,), ZSLOT, jnp.int32),
                           mask=m)

    C = process_if(C, C < F)
    F = fire_if(F, (Wp - F * BATCH_R >= BATCH_R) & (F < C + 2))
    C = process_if(C, C < F)
    C = process_if(C, C < F)

    for back in (2, 1):
        @pl.when((C >= back) & ((C - back) % 2 == 0))
        def _():
            sca_desc(C - back, 0, ssem_a).wait()

        @pl.when((C >= back) & ((C - back) % 2 == 1))
        def _():
            sca_desc(C - back, 1, ssem_b).wait()

    plsc.subcore_barrier()

    NP3 = CPT // PP
    PW = PP * MAX_G

    @pl.loop(0, CPT * MAX_G // L)
    def _(i):
        g = mat_v[pl.ds(sid * CPT * MAX_G + i * L, L)]
        s = _decode(map16, jnp.maximum(g, 0))
        slots3[pl.ds(i * L, L)] = jnp.where(g >= 0, s, ZSLOT)

    def p3_gat(h, par):
        return pltpu.make_async_copy(
            acc.at[slots3.at[pl.ds(h * PW, PW)]],
            gbuf.at[pl.ds(par * PW, PW)], g3sem)

    def p3_out(h, par):
        return pltpu.make_async_copy(
            pooled_v.at[pl.ds(par * PP, PP)],
            out_hbm.at[cid, pl.ds(sid * CPT + h * PP, PP)], psem)

    p3_gat(0, 0).start()
    p3_gat(1, 1).start()

    @pl.loop(0, NP3)
    def _(h):
        par = h % 2
        p3_gat(h, par).wait()

        @pl.when(h >= 2)
        def _():
            p3_out(h - 2, par).wait()

        @pl.loop(0, PP)
        def _(cc):
            b5 = par * PW + cc * MAX_G
            for q in range(EMB // L):
                sl = pl.ds(q * L, L)
                ssum = gbuf[b5, sl]
                for j in range(1, MAX_G):
                    ssum = ssum + gbuf[b5 + j, sl]
                pooled_v[par * PP + cc, sl] = ssum

        p3_out(h, par).start()

        @pl.when(h + 2 < NP3)
        def _():
            p3_gat(h + 2, par).start()

    p3_out(NP3 - 2, 0).wait()
    p3_out(NP3 - 1, 1).wait()


_sc_mesh = plsc.VectorSubcoreMesh(core_axis_name="c", subcore_axis_name="s")
_sc_params = pltpu.CompilerParams(needs_layout_passes=False,
                                  use_tc_tiling_on_sc=False)

_agg_pool = pl.kernel(
    _sc_body,
    out_type=jax.ShapeDtypeStruct((NC, N_COND, EMB), jnp.float32),
    mesh=_sc_mesh,
    compiler_params=_sc_params,
    scratch_types=[
        pltpu.VMEM((MAPW,), jnp.int32),
        pltpu.VMEM((N_COND * MAX_G,), jnp.int32),
        pltpu.VMEM((3, CHUNK), jnp.int32),
        pltpu.VMEM((3, CHUNK), jnp.int32),
        pltpu.VMEM((3, CHUNK), jnp.float32),
        pltpu.VMEM((RING,), jnp.int32),
        pltpu.VMEM((RING,), jnp.float32),
        pltpu.VMEM((RING,), jnp.int32),
        pltpu.VMEM((2 * BATCH_R, EMB), jnp.float32),
        pltpu.VMEM((2, BATCH_R), jnp.int32),
        pltpu.VMEM((CPT * MAX_G,), jnp.int32),
        pltpu.VMEM((2 * PP * MAX_G, EMB), jnp.float32),
        pltpu.VMEM((2 * PP, EMB), jnp.float32),
        pltpu.VMEM_SHARED((ACC_ROWS, EMB), jnp.float32),
        pltpu.SemaphoreType.DMA,
        pltpu.SemaphoreType.DMA,
        pltpu.SemaphoreType.DMA,
        pltpu.SemaphoreType.DMA,
        pltpu.SemaphoreType.DMA,
        pltpu.SemaphoreType.DMA,
        pltpu.SemaphoreType.DMA,
    ],
)


def _tc_body(pp_ref, gnn_ref, w1_ref, b1_ref, w2_ref, b2_ref, mask_ref,
             o_ref):
    p = pp_ref[0] + pp_ref[1]
    summed = jnp.dot(p, gnn_ref[...], preferred_element_type=jnp.float32)
    h = jnp.maximum(
        jnp.dot(summed, w1_ref[...], preferred_element_type=jnp.float32)
        + b1_ref[...], 0.0)
    h = jnp.maximum(
        jnp.dot(h, w2_ref[...], preferred_element_type=jnp.float32)
        + b2_ref[...], 0.0)
    ng = jnp.sum(mask_ref[...], axis=1, keepdims=True)
    o_ref[...] = jnp.where(ng == 0.0, 0.0, jnp.where(ng == 1.0, summed, h))


_mlp = pl.pallas_call(
    _tc_body,
    out_shape=jax.ShapeDtypeStruct((N_COND, EMB), jnp.float32),
)


def _gat_body(tab_hbm, idx_hbm, out_hbm, idx_v, row_v, sem):
    wid = lax.axis_index("c") * NS + lax.axis_index("s")
    base = wid * (BATCH // NW)
    pltpu.sync_copy(idx_hbm.at[pl.ds(base, BATCH // NW)], idx_v)
    pltpu.async_copy(tab_hbm.at[idx_v], row_v, sem).wait()
    pltpu.sync_copy(row_v, out_hbm.at[pl.ds(base, BATCH // NW)])


_expand = pl.kernel(
    _gat_body,
    out_type=jax.ShapeDtypeStruct((BATCH, EMB), jnp.float32),
    mesh=_sc_mesh,
    compiler_params=_sc_params,
    scratch_types=[
        pltpu.VMEM((BATCH // NW,), jnp.int32),
        pltpu.VMEM((BATCH // NW, EMB), jnp.float32),
        pltpu.SemaphoreType.DMA,
    ],
)


def kernel(inputs, pert_embedding, gnn_kernel, mlp_w1, mlp_b1, mlp_w2, mlp_b2,
           adj_rows, adj_cols, adj_vals, cond_gene_matrix, cond_gene_mask):
    mat_flat = cond_gene_matrix.reshape(-1)
    zmap = jnp.zeros((MAPW,), jnp.int32)
    zrows = jnp.zeros((ACC_STRIPE, EMB), jnp.float32)

    pooled_partial = _agg_pool(adj_rows, adj_cols, adj_vals, mat_flat,
                               pert_embedding, zmap, zrows)
    mask8 = jnp.pad(cond_gene_mask, ((0, 0), (0, 3)))
    out_cond = _mlp(pooled_partial, gnn_kernel, mlp_w1,
                    mlp_b1.reshape(1, EMB), mlp_w2, mlp_b2.reshape(1, EMB),
                    mask8)
    return _expand(out_cond, inputs.astype(jnp.int32))
